# Initial kernel scaffold; baseline (speedup 1.0000x reference)
import jax
import jax.numpy as jnp
from jax.experimental import pallas as pl


def kernel(x, y, anchors):
    raise NotImplementedError("write your pallas kernel here")


# speedup vs baseline: 211.1053x; 211.1053x over previous
import jax
import jax.numpy as jnp
from jax import lax
from jax.experimental import pallas as pl

_N = 5000
_TOP_K = 100
_SCALE = 192.0
_IOU_T = 0.3
_R, _C = 8, 640
_PAD = _R * _C - _N
_OUT_ROWS = 104
_BIG = 2 ** 30


def _nms_kernel(logit_ref, y0_ref, y1_ref, y2_ref, y3_ref, y4_ref, y5_ref,
                y8_ref, y9_ref, ax_ref, ay_ref, out_ref):
    ax = ax_ref[...]
    ay = ay_ref[...]

    score = jax.nn.sigmoid(logit_ref[...])
    cx = y0_ref[...] / _SCALE + ax
    cy = y1_ref[...] / _SCALE + ay
    w = y2_ref[...] / _SCALE
    h = y3_ref[...] / _SCALE
    kp0x = y4_ref[...] / _SCALE + ax
    kp0y = y5_ref[...] / _SCALE + ay
    kp2x = y8_ref[...] / _SCALE + ax
    kp2y = y9_ref[...] / _SCALE + ay

    half_w = w * 0.5
    half_h = h * 0.5
    x1 = cx - half_w
    x2 = cx + half_w
    yl = cy - half_h
    yh = cy + half_h
    area = (x2 - x1) * (yh - yl)

    li = (lax.broadcasted_iota(jnp.int32, (_R, _C), 0) * _C
          + lax.broadcasted_iota(jnp.int32, (_R, _C), 1))
    out_col = lax.broadcasted_iota(jnp.int32, (1, 8), 1)

    def gather(a, k):
        return jnp.sum(jnp.where(li == k, a, 0.0))

    s_init = jnp.where(li < _N, score, -1.0)
    k0 = jnp.min(jnp.where(s_init == jnp.max(s_init), li, _BIG))

    def body(t, s_live):
        m = jnp.max(s_live)
        has = m >= 0.0
        k_sel = jnp.min(jnp.where(s_live == m, li, _BIG))
        k = jnp.where(has, k_sel, k0)

        ks = gather(score, k)
        kcx = gather(cx, k)
        kcy = gather(cy, k)
        kw = gather(w, k)
        kk0x = gather(kp0x, k)
        kk0y = gather(kp0y, k)
        kk2x = gather(kp2x, k)
        kk2y = gather(kp2y, k)
        kx1 = gather(x1, k)
        kx2 = gather(x2, k)
        kyl = gather(yl, k)
        kyh = gather(yh, k)
        karea = gather(area, k)

        xx1 = jnp.maximum(x1, kx1)
        yy1 = jnp.maximum(yl, kyl)
        xx2 = jnp.minimum(x2, kx2)
        yy2 = jnp.minimum(yh, kyh)
        inter = jnp.maximum(xx2 - xx1, 0.0) * jnp.maximum(yy2 - yy1, 0.0)
        iou = inter / (area + karea - inter)
        dead = (iou > _IOU_T) | (li == k)
        s_new = jnp.where(dead, -1.0, s_live)
        s_live = jnp.where(has, s_new, s_live)

        vals = (ks, kcx, kcy, kw, kk0x, kk0y, kk2x, kk2y)
        row = jnp.zeros((1, 8), jnp.float32)
        for j, v in enumerate(vals):
            row = row + jnp.where(out_col == j, v, 0.0)
        out_ref[pl.ds(t, 1), :] = row
        return s_live

    lax.fori_loop(0, _TOP_K, body, s_init)


def kernel(x, y, anchors):
    xf = x[0, :, 0]
    yy = y[0]

    def prep(a):
        return jnp.pad(a, (0, _PAD)).reshape(_R, _C)

    args = [prep(xf),
            prep(yy[:, 0]), prep(yy[:, 1]), prep(yy[:, 2]), prep(yy[:, 3]),
            prep(yy[:, 4]), prep(yy[:, 5]), prep(yy[:, 8]), prep(yy[:, 9]),
            prep(anchors[:, 0]), prep(anchors[:, 1])]

    out = pl.pallas_call(
        _nms_kernel,
        out_shape=jax.ShapeDtypeStruct((_OUT_ROWS, 8), jnp.float32),
    )(*args)
    return out[:_TOP_K]
